# Initial kernel scaffold; baseline (speedup 1.0000x reference)
import jax
import jax.numpy as jnp
from jax.experimental import pallas as pl


def kernel(gene_id, count_id, gene_table, count_table):
    raise NotImplementedError("write your pallas kernel here")


# speedup vs baseline: 6.3582x; 6.3582x over previous
import functools

import jax
import jax.numpy as jnp
from jax import lax
from jax.experimental import pallas as pl
from jax.experimental.pallas import tpu as pltpu
from jax.experimental.pallas import tpu_sc as plsc

_info = plsc.get_sparse_core_info()
_NC = _info.num_cores
_NS = _info.num_subcores
_NW = _NC * _NS

_IDXC = 128
_KSUB = 4
_CHUNK = _IDXC * _KSUB
_NBUF = 2


def _make_body(n_rows, emb, n_w, groups):
    rows_w = n_w // _IDXC

    def body(gidx_hbm, cidx_hbm, gtab_hbm, ctab_hbm, out_hbm,
             gidx_v, cidx_v, rows_v, gsem, asem, osem):
        wid = lax.axis_index("s") * _NC + lax.axis_index("c")
        base = wid * n_w
        irow0 = wid * rows_w

        pltpu.sync_copy(gidx_hbm.at[pl.ds(irow0, rows_w)], gidx_v)
        pltpu.sync_copy(cidx_hbm.at[pl.ds(irow0, rows_w)], cidx_v)

        def group(g, carry):
            for b in range(_NBUF):
                s = g * _NBUF + b

                @pl.when(g >= 1)
                def _wait_prev():
                    pltpu.make_async_copy(
                        rows_v.at[b],
                        out_hbm.at[pl.ds(base, _CHUNK)],
                        osem).wait()

                cps = []
                for j in range(_KSUB):
                    cps.append(pltpu.async_copy(
                        gtab_hbm.at[gidx_v.at[s * _KSUB + j]],
                        rows_v.at[b, pl.ds(j * _IDXC, _IDXC)],
                        gsem))
                for cp in cps:
                    cp.wait()

                cps = []
                for j in range(_KSUB):
                    cps.append(pltpu.async_copy(
                        ctab_hbm.at[cidx_v.at[s * _KSUB + j]],
                        rows_v.at[b, pl.ds(j * _IDXC, _IDXC)],
                        asem, add=True))
                for cp in cps:
                    cp.wait()

                pltpu.async_copy(
                    rows_v.at[b],
                    out_hbm.at[pl.ds(base + s * _CHUNK, _CHUNK)],
                    osem)
            return carry

        lax.fori_loop(0, groups, group, 0)

        for b in range(_NBUF):
            pltpu.make_async_copy(
                rows_v.at[b],
                out_hbm.at[pl.ds(base, _CHUNK)],
                osem).wait()

    return body


@functools.partial(jax.jit, static_argnames=())
def _embedding_sum(gidx, cidx, gtab, ctab):
    n_idx_rows, idxc = gidx.shape
    n_rows = n_idx_rows * idxc
    emb = gtab.shape[1]
    n_w = n_rows // _NW
    groups = n_w // (_CHUNK * _NBUF)
    rows_w = n_w // _IDXC

    body = _make_body(n_rows, emb, n_w, groups)
    call = pl.kernel(
        body,
        out_type=jax.ShapeDtypeStruct((n_rows, emb), jnp.float32),
        scratch_types=[
            pltpu.VMEM((rows_w, _IDXC), jnp.int32),
            pltpu.VMEM((rows_w, _IDXC), jnp.int32),
            pltpu.VMEM((_NBUF, _CHUNK, emb), jnp.float32),
            pltpu.SemaphoreType.DMA,
            pltpu.SemaphoreType.DMA,
            pltpu.SemaphoreType.DMA,
        ],
        mesh=plsc.VectorSubcoreMesh(core_axis_name="c", subcore_axis_name="s"),
        compiler_params=pltpu.CompilerParams(use_tc_tiling_on_sc=False),
    )
    return call(gidx, cidx, gtab, ctab)


def kernel(gene_id, count_id, gene_table, count_table):
    b, s = gene_id.shape
    n = b * s
    gidx = gene_id.reshape(n // _IDXC, _IDXC).astype(jnp.int32)
    cidx = count_id.reshape(n // _IDXC, _IDXC).astype(jnp.int32)
    out = _embedding_sum(gidx, cidx, gene_table, count_table)
    return out.reshape(b, s, gene_table.shape[1])


# speedup vs baseline: 6.3988x; 1.0064x over previous
import functools

import jax
import jax.numpy as jnp
from jax import lax
from jax.experimental import pallas as pl
from jax.experimental.pallas import tpu as pltpu
from jax.experimental.pallas import tpu_sc as plsc

_info = plsc.get_sparse_core_info()
_NC = _info.num_cores
_NS = _info.num_subcores
_NW = _NC * _NS

_IDXC = 128
_KSUB = 2
_CHUNK = _IDXC * _KSUB
_NBUF = 4


def _make_body(n_w, steps):
    rows_w = n_w // _IDXC

    def body(gidx_hbm, cidx_hbm, gtab_hbm, ctab_hbm, out_hbm,
             gidx_v, cidx_v, rows_v,
             gsem0, gsem1, asem0, asem1, osem0, osem1, osem2, osem3):
        gsems = (gsem0, gsem1)
        asems = (asem0, asem1)
        osems = (osem0, osem1, osem2, osem3)
        wid = lax.axis_index("s") * _NC + lax.axis_index("c")
        base = wid * n_w
        irow0 = wid * rows_w

        pltpu.sync_copy(gidx_hbm.at[pl.ds(irow0, rows_w)], gidx_v)
        pltpu.sync_copy(cidx_hbm.at[pl.ds(irow0, rows_w)], cidx_v)

        def gene_start(s, b, gsem):
            for j in range(_KSUB):
                pltpu.async_copy(
                    gtab_hbm.at[gidx_v.at[s * _KSUB + j]],
                    rows_v.at[b, pl.ds(j * _IDXC, _IDXC)],
                    gsem)

        def gene_wait(b, gsem):
            for j in range(_KSUB):
                pltpu.make_async_copy(
                    gtab_hbm.at[gidx_v.at[j]],
                    rows_v.at[b, pl.ds(j * _IDXC, _IDXC)],
                    gsem).wait()

        def add_start(s, b, asem):
            for j in range(_KSUB):
                pltpu.async_copy(
                    ctab_hbm.at[cidx_v.at[s * _KSUB + j]],
                    rows_v.at[b, pl.ds(j * _IDXC, _IDXC)],
                    asem, add=True)

        def add_wait(b, asem):
            for j in range(_KSUB):
                pltpu.make_async_copy(
                    ctab_hbm.at[cidx_v.at[j]],
                    rows_v.at[b, pl.ds(j * _IDXC, _IDXC)],
                    asem).wait()

        def write_start(s, b, osem):
            pltpu.async_copy(
                rows_v.at[b],
                out_hbm.at[pl.ds(base + s * _CHUNK, _CHUNK)],
                osem)

        def write_wait(b, osem):
            pltpu.make_async_copy(
                rows_v.at[b],
                out_hbm.at[pl.ds(base, _CHUNK)],
                osem).wait()

        gene_start(0, 0, gsems[0])

        def group(g, carry):
            for q in range(_NBUF):
                s = g * _NBUF + q
                b = q
                p = q % 2
                bn = (q + 1) % _NBUF

                @pl.when(s + 1 < steps)
                def _pref(s=s, bn=bn, p=p):
                    @pl.when(s + 1 >= _NBUF)
                    def _free():
                        write_wait(bn, osems[bn])
                    gene_start(s + 1, bn, gsems[1 - p])

                gene_wait(b, gsems[p])
                add_start(s, b, asems[p])

                @pl.when(s >= 1)
                def _retire(s=s, b=b, p=p):
                    bp = (b - 1) % _NBUF
                    add_wait(bp, asems[1 - p])
                    write_start(s - 1, bp, osems[bp])
            return carry

        lax.fori_loop(0, steps // _NBUF, group, 0)

        last = steps - 1
        bl = last % _NBUF
        add_wait(bl, asems[last % 2])
        write_start(last, bl, osems[bl])
        for b in range(_NBUF):
            write_wait(b, osems[b])

    return body


@functools.partial(jax.jit, static_argnames=())
def _embedding_sum(gidx, cidx, gtab, ctab):
    n_idx_rows, idxc = gidx.shape
    n_rows = n_idx_rows * idxc
    emb = gtab.shape[1]
    n_w = n_rows // _NW
    steps = n_w // _CHUNK
    rows_w = n_w // _IDXC

    body = _make_body(n_w, steps)
    call = pl.kernel(
        body,
        out_type=jax.ShapeDtypeStruct((n_rows, emb), jnp.float32),
        scratch_types=[
            pltpu.VMEM((rows_w, _IDXC), jnp.int32),
            pltpu.VMEM((rows_w, _IDXC), jnp.int32),
            pltpu.VMEM((_NBUF, _CHUNK, emb), jnp.float32),
        ] + [pltpu.SemaphoreType.DMA] * 8,
        mesh=plsc.VectorSubcoreMesh(core_axis_name="c", subcore_axis_name="s"),
        compiler_params=pltpu.CompilerParams(use_tc_tiling_on_sc=False),
    )
    return call(gidx, cidx, gtab, ctab)


def kernel(gene_id, count_id, gene_table, count_table):
    b, s = gene_id.shape
    n = b * s
    gidx = gene_id.reshape(n // _IDXC, _IDXC).astype(jnp.int32)
    cidx = count_id.reshape(n // _IDXC, _IDXC).astype(jnp.int32)
    out = _embedding_sum(gidx, cidx, gene_table, count_table)
    return out.reshape(b, s, gene_table.shape[1])


# speedup vs baseline: 6.4077x; 1.0014x over previous
import functools

import jax
import jax.numpy as jnp
from jax import lax
from jax.experimental import pallas as pl
from jax.experimental.pallas import tpu as pltpu
from jax.experimental.pallas import tpu_sc as plsc

_info = plsc.get_sparse_core_info()
_NC = _info.num_cores
_NS = _info.num_subcores
_NW = _NC * _NS

_NBUF = 4


def _make_body(batch, seq, emb, steps):
    n_w = steps * seq
    splits = []
    off = 0
    while off < seq:
        ln = min(128, seq - off)
        splits.append((off, ln))
        off += ln

    def body(gidx_hbm, cidx_hbm, gtab_hbm, ctab_hbm, out_hbm,
             gidx_v, cidx_v, rows_v,
             gsem0, gsem1, asem0, asem1, osem0, osem1, osem2, osem3):
        gsems = (gsem0, gsem1)
        asems = (asem0, asem1)
        osems = (osem0, osem1, osem2, osem3)
        wid = lax.axis_index("s") * _NC + lax.axis_index("c")
        base_b = wid * steps
        flat0 = wid * n_w

        pltpu.sync_copy(gidx_hbm.at[pl.ds(flat0, n_w)], gidx_v)
        pltpu.sync_copy(cidx_hbm.at[pl.ds(flat0, n_w)], cidx_v)

        def gene_start(s, b, gsem):
            for off, ln in splits:
                pltpu.async_copy(
                    gtab_hbm.at[gidx_v.at[pl.ds(s * seq + off, ln)]],
                    rows_v.at[b, pl.ds(off, ln)],
                    gsem)

        def gene_wait(b, gsem):
            for off, ln in splits:
                pltpu.make_async_copy(
                    gtab_hbm.at[gidx_v.at[pl.ds(0, ln)]],
                    rows_v.at[b, pl.ds(off, ln)],
                    gsem).wait()

        def add_start(s, b, asem):
            for off, ln in splits:
                pltpu.async_copy(
                    ctab_hbm.at[cidx_v.at[pl.ds(s * seq + off, ln)]],
                    rows_v.at[b, pl.ds(off, ln)],
                    asem, add=True)

        def add_wait(b, asem):
            for off, ln in splits:
                pltpu.make_async_copy(
                    ctab_hbm.at[cidx_v.at[pl.ds(0, ln)]],
                    rows_v.at[b, pl.ds(off, ln)],
                    asem).wait()

        def write_start(s, b, osem):
            pltpu.async_copy(rows_v.at[b], out_hbm.at[base_b + s], osem)

        def write_wait(b, osem):
            pltpu.make_async_copy(rows_v.at[b], out_hbm.at[0], osem).wait()

        gene_start(0, 0, gsems[0])

        def group(g, carry):
            for q in range(_NBUF):
                s = g * _NBUF + q
                b = q
                p = q % 2
                bn = (q + 1) % _NBUF

                @pl.when(s + 1 < steps)
                def _pref(s=s, bn=bn, p=p):
                    @pl.when(s + 1 >= _NBUF)
                    def _free():
                        write_wait(bn, osems[bn])
                    gene_start(s + 1, bn, gsems[1 - p])

                gene_wait(b, gsems[p])
                add_start(s, b, asems[p])

                @pl.when(s >= 1)
                def _retire(s=s, b=b, p=p):
                    bp = (b - 1) % _NBUF
                    add_wait(bp, asems[1 - p])
                    write_start(s - 1, bp, osems[bp])
            return carry

        lax.fori_loop(0, steps // _NBUF, group, 0)

        last = steps - 1
        bl = last % _NBUF
        add_wait(bl, asems[last % 2])
        write_start(last, bl, osems[bl])
        for b in range(_NBUF):
            write_wait(b, osems[b])

    return body


@functools.partial(jax.jit, static_argnums=(4, 5))
def _embedding_sum(gidx, cidx, gtab, ctab, batch, seq):
    emb = gtab.shape[1]
    steps = batch // _NW
    n_w = steps * seq

    body = _make_body(batch, seq, emb, steps)
    call = pl.kernel(
        body,
        out_type=jax.ShapeDtypeStruct((batch, seq, emb), jnp.float32),
        scratch_types=[
            pltpu.VMEM((n_w,), jnp.int32),
            pltpu.VMEM((n_w,), jnp.int32),
            pltpu.VMEM((_NBUF, seq, emb), jnp.float32),
        ] + [pltpu.SemaphoreType.DMA] * 8,
        mesh=plsc.VectorSubcoreMesh(core_axis_name="c", subcore_axis_name="s"),
        compiler_params=pltpu.CompilerParams(use_tc_tiling_on_sc=False),
    )
    return call(gidx, cidx, gtab, ctab)


def kernel(gene_id, count_id, gene_table, count_table):
    b, s = gene_id.shape
    gidx = gene_id.reshape(-1).astype(jnp.int32)
    cidx = count_id.reshape(-1).astype(jnp.int32)
    return _embedding_sum(gidx, cidx, gene_table, count_table, b, s)


# speedup vs baseline: 10.6299x; 1.6589x over previous
import functools

import jax
import jax.numpy as jnp
from jax import lax
from jax.experimental import pallas as pl
from jax.experimental.pallas import tpu as pltpu
from jax.experimental.pallas import tpu_sc as plsc

_info = plsc.get_sparse_core_info()
_NC = _info.num_cores
_NS = _info.num_subcores
_NW = _NC * _NS

_IDXC = 128
_CHUNK = _IDXC
_GRP = 4
_NWIDE = 4
_PADW = 128


def _make_body(n_w, steps, emb, cvocab):
    rows_w = n_w // _IDXC

    def body(gidx_hbm, cidx_hbm, gtab_hbm, ctab_hbm, out_hbm,
             gidx_v, cidx_v, wide_v, comp_v, ctab_sh,
             gsem0, gsem1, asem0, asem1, osem0, osem1, isem0, isem1):
        gsems = (gsem0, gsem1)
        asems = (asem0, asem1)
        osems = (osem0, osem1)
        isems = (isem0, isem1)
        sid = lax.axis_index("s")
        wid = sid * _NC + lax.axis_index("c")
        base = wid * n_w
        irow0 = wid * rows_w

        @pl.when(sid == 0)
        def _stage():
            pltpu.sync_copy(ctab_hbm, ctab_sh)
        plsc.subcore_barrier()

        def idx_start(g, ib):
            pltpu.async_copy(gidx_hbm.at[pl.ds(irow0 + g * _GRP, _GRP)],
                             gidx_v.at[ib], isems[ib])
            pltpu.async_copy(cidx_hbm.at[pl.ds(irow0 + g * _GRP, _GRP)],
                             cidx_v.at[ib], isems[ib])

        def idx_wait(ib):
            pltpu.make_async_copy(gidx_hbm.at[pl.ds(irow0, _GRP)],
                                  gidx_v.at[ib], isems[ib]).wait()
            pltpu.make_async_copy(cidx_hbm.at[pl.ds(irow0, _GRP)],
                                  cidx_v.at[ib], isems[ib]).wait()

        def gene_start(ib, q, w, gsem):
            pltpu.async_copy(gtab_hbm.at[gidx_v.at[ib, q]],
                             wide_v.at[w], gsem)

        def gene_wait(w, gsem):
            pltpu.make_async_copy(gtab_hbm.at[gidx_v.at[0, 0]],
                                  wide_v.at[w], gsem).wait()

        def add_start(ib, q, w, asem):
            pltpu.async_copy(ctab_sh.at[cidx_v.at[ib, q]],
                             wide_v.at[w], asem, add=True)

        def add_wait(w, asem):
            pltpu.make_async_copy(ctab_sh.at[cidx_v.at[0, 0]],
                                  wide_v.at[w], asem).wait()

        def compact(w, c):
            def row(r, carry):
                for k in range(0, emb, 16):
                    comp_v[c, r, pl.ds(k, 16)] = wide_v[w, r, pl.ds(k, 16)]
                return carry
            lax.fori_loop(0, _CHUNK, row, 0)

        def write_start(s, c, osem):
            pltpu.async_copy(comp_v.at[c],
                             out_hbm.at[pl.ds(base + s * _CHUNK, _CHUNK)],
                             osem)

        def write_wait(c, osem):
            pltpu.make_async_copy(comp_v.at[c],
                                  out_hbm.at[pl.ds(base, _CHUNK)],
                                  osem).wait()

        idx_start(0, 0)
        idx_wait(0)
        gene_start(0, 0, 0, gsems[0])

        ngroups = steps // _GRP

        def group(g2, carry):
            for gpar in range(2):
              g = g2 * 2 + gpar
              for q in range(_GRP):
                s = g * _GRP + q
                w = q % _NWIDE
                p = q % 2
                c = q % 2

                if q == 1:
                    @pl.when(g + 1 < ngroups)
                    def _pref_idx(g=g, gpar=gpar):
                        idx_start(g + 1, (gpar + 1) % 2)

                @pl.when(s + 1 < steps)
                def _pref(s=s, q=q, w=w, p=p, gpar=gpar):
                    qn = (q + 1) % _GRP
                    if qn == 0:
                        idx_wait((gpar + 1) % 2)
                    ibn = (gpar + (1 if qn == 0 else 0)) % 2
                    gene_start(ibn, qn, (w + 1) % _NWIDE, gsems[1 - p])

                gene_wait(w, gsems[p])
                add_start(gpar, q, w, asems[p])

                @pl.when(s >= 1)
                def _retire(s=s, w=w, p=p, c=c):
                    wp = (w - 1) % _NWIDE
                    add_wait(wp, asems[1 - p])
                    @pl.when(s >= 3)
                    def _free():
                        write_wait(1 - c, osems[1 - c])
                    compact(wp, 1 - c)
                    write_start(s - 1, 1 - c, osems[1 - c])
            return carry

        lax.fori_loop(0, steps // (2 * _GRP), group, 0)

        last = steps - 1
        wl = last % _NWIDE
        cl = last % 2
        add_wait(wl, asems[last % 2])
        write_wait(cl, osems[cl])
        compact(wl, cl)
        write_start(last, cl, osems[cl])
        for c in range(2):
            write_wait(c, osems[c])

    return body


@functools.partial(jax.jit, static_argnums=())
def _embedding_sum(gidx, cidx, gtab, ctab):
    n_idx_rows, idxc = gidx.shape
    n_rows = n_idx_rows * idxc
    emb = 64
    cvocab = ctab.shape[0]
    n_w = n_rows // _NW
    steps = n_w // _CHUNK

    body = _make_body(n_w, steps, emb, cvocab)
    call = pl.kernel(
        body,
        out_type=jax.ShapeDtypeStruct((n_rows, emb), jnp.float32),
        scratch_types=[
            pltpu.VMEM((2, _GRP, _IDXC), jnp.int32),
            pltpu.VMEM((2, _GRP, _IDXC), jnp.int32),
            pltpu.VMEM((_NWIDE, _CHUNK, _PADW), jnp.float32),
            pltpu.VMEM((2, _CHUNK, emb), jnp.float32),
            pltpu.VMEM_SHARED((cvocab, _PADW), jnp.float32),
        ] + [pltpu.SemaphoreType.DMA] * 8,
        mesh=plsc.VectorSubcoreMesh(core_axis_name="c", subcore_axis_name="s"),
        compiler_params=pltpu.CompilerParams(use_tc_tiling_on_sc=True),
    )
    return call(gidx, cidx, gtab, ctab)


def kernel(gene_id, count_id, gene_table, count_table):
    b, s = gene_id.shape
    n = b * s
    emb = gene_table.shape[1]
    gidx = gene_id.reshape(n // _IDXC, _IDXC).astype(jnp.int32)
    cidx = count_id.reshape(n // _IDXC, _IDXC).astype(jnp.int32)
    gtab = jnp.pad(gene_table, ((0, 0), (0, _PADW - emb)))
    ctab = jnp.pad(count_table, ((0, 0), (0, _PADW - emb)))
    out = _embedding_sum(gidx, cidx, gtab, ctab)
    return out.reshape(b, s, emb)


# speedup vs baseline: 10.6331x; 1.0003x over previous
import functools

import jax
import jax.numpy as jnp
from jax import lax
from jax.experimental import pallas as pl
from jax.experimental.pallas import tpu as pltpu
from jax.experimental.pallas import tpu_sc as plsc

_info = plsc.get_sparse_core_info()
_NC = _info.num_cores
_NS = _info.num_subcores
_NW = _NC * _NS

_IDXC = 128
_CHUNK = _IDXC
_GRP = 4
_NWIDE = 4
_PADW = 128


def _make_body(n_w, steps, emb, cvocab):
    rows_w = n_w // _IDXC

    def body(gidx_hbm, cidx_hbm, gtab_hbm, ctab_hbm, out_hbm,
             gidx_v, cidx_v, wide_v, comp_v, ctab_sh,
             gsem0, gsem1, asem0, asem1, osem0, osem1, isem0, isem1):
        gsems = (gsem0, gsem1)
        asems = (asem0, asem1)
        osems = (osem0, osem1)
        isems = (isem0, isem1)
        sid = lax.axis_index("s")
        wid = sid * _NC + lax.axis_index("c")
        base = wid * n_w
        irow0 = wid * rows_w

        @pl.when(sid == 0)
        def _stage():
            pltpu.sync_copy(ctab_hbm, ctab_sh)
        plsc.subcore_barrier()

        def idx_start(g, ib):
            pltpu.async_copy(gidx_hbm.at[pl.ds(irow0 + g * _GRP, _GRP)],
                             gidx_v.at[ib], isems[ib])
            pltpu.async_copy(cidx_hbm.at[pl.ds(irow0 + g * _GRP, _GRP)],
                             cidx_v.at[ib], isems[ib])

        def idx_wait(ib):
            pltpu.make_async_copy(gidx_hbm.at[pl.ds(irow0, _GRP)],
                                  gidx_v.at[ib], isems[ib]).wait()
            pltpu.make_async_copy(cidx_hbm.at[pl.ds(irow0, _GRP)],
                                  cidx_v.at[ib], isems[ib]).wait()

        def gene_start(ib, q, w, gsem):
            pltpu.async_copy(gtab_hbm.at[gidx_v.at[ib, q]],
                             wide_v.at[w], gsem)

        def gene_wait(w, gsem):
            pltpu.make_async_copy(gtab_hbm.at[gidx_v.at[0, 0]],
                                  wide_v.at[w], gsem).wait()

        def add_start(ib, q, w, asem):
            pltpu.async_copy(ctab_sh.at[cidx_v.at[ib, q]],
                             wide_v.at[w], asem, add=True)

        def add_wait(w, asem):
            pltpu.make_async_copy(ctab_sh.at[cidx_v.at[0, 0]],
                                  wide_v.at[w], asem).wait()

        def compact(w, c):
            def row(r, carry):
                for k in range(0, emb, 16):
                    comp_v[c, r, pl.ds(k, 16)] = wide_v[w, r, pl.ds(k, 16)]
                return carry
            lax.fori_loop(0, _CHUNK, row, 0)

        def write_start(s, c, osem):
            pltpu.async_copy(comp_v.at[c],
                             out_hbm.at[pl.ds(base + s * _CHUNK, _CHUNK)],
                             osem)

        def write_wait(c, osem):
            pltpu.make_async_copy(comp_v.at[c],
                                  out_hbm.at[pl.ds(base, _CHUNK)],
                                  osem).wait()

        idx_start(0, 0)
        idx_wait(0)
        gene_start(0, 0, 0, gsems[0])

        ngroups = steps // _GRP

        def group(g2, carry):
            for gpar in range(2):
              g = g2 * 2 + gpar
              for q in range(_GRP):
                s = g * _GRP + q
                w = q % _NWIDE
                p = q % 2
                c = q % 2

                if q == 1:
                    @pl.when(g + 1 < ngroups)
                    def _pref_idx(g=g, gpar=gpar):
                        idx_start(g + 1, (gpar + 1) % 2)

                @pl.when(s + 1 < steps)
                def _pref(s=s, q=q, w=w, p=p, gpar=gpar):
                    qn = (q + 1) % _GRP
                    if qn == 0:
                        idx_wait((gpar + 1) % 2)
                    ibn = (gpar + (1 if qn == 0 else 0)) % 2
                    gene_start(ibn, qn, (w + 1) % _NWIDE, gsems[1 - p])

                gene_wait(w, gsems[p])
                add_start(gpar, q, w, asems[p])

                @pl.when(s >= 1)
                def _retire(s=s, w=w, p=p, c=c):
                    wp = (w - 1) % _NWIDE
                    add_wait(wp, asems[1 - p])
                    @pl.when(s >= 3)
                    def _free():
                        write_wait(1 - c, osems[1 - c])
                    compact(wp, 1 - c)
                    write_start(s - 1, 1 - c, osems[1 - c])
            return carry

        lax.fori_loop(0, steps // (2 * _GRP), group, 0)

        last = steps - 1
        wl = last % _NWIDE
        cl = last % 2
        add_wait(wl, asems[last % 2])
        write_wait(cl, osems[cl])
        compact(wl, cl)
        write_start(last, cl, osems[cl])
        for c in range(2):
            write_wait(c, osems[c])

    return body


@functools.partial(jax.jit, static_argnums=())
def _embedding_sum(gidx, cidx, gtab, ctab):
    n_idx_rows, idxc = gidx.shape
    n_rows = n_idx_rows * idxc
    emb = 64
    cvocab = ctab.shape[0]
    n_w = n_rows // _NW
    steps = n_w // _CHUNK

    body = _make_body(n_w, steps, emb, cvocab)
    call = pl.kernel(
        body,
        out_type=jax.ShapeDtypeStruct((n_rows, emb), jnp.float32),
        scratch_types=[
            pltpu.VMEM((2, _GRP, _IDXC), jnp.int32),
            pltpu.VMEM((2, _GRP, _IDXC), jnp.int32),
            pltpu.VMEM((_NWIDE, _CHUNK, _PADW), jnp.float32),
            pltpu.VMEM((2, _CHUNK, emb), jnp.float32),
            pltpu.VMEM_SHARED((cvocab, _PADW), jnp.float32),
        ] + [pltpu.SemaphoreType.DMA] * 8,
        mesh=plsc.VectorSubcoreMesh(core_axis_name="c", subcore_axis_name="s"),
        compiler_params=pltpu.CompilerParams(use_tc_tiling_on_sc=True,
                                             needs_layout_passes=False),
    )
    return call(gidx, cidx, gtab, ctab)


def kernel(gene_id, count_id, gene_table, count_table):
    b, s = gene_id.shape
    n = b * s
    emb = gene_table.shape[1]
    gidx = gene_id.reshape(n // _IDXC, _IDXC).astype(jnp.int32)
    cidx = count_id.reshape(n // _IDXC, _IDXC).astype(jnp.int32)
    gtab = jnp.pad(gene_table, ((0, 0), (0, _PADW - emb)))
    ctab = jnp.pad(count_table, ((0, 0), (0, _PADW - emb)))
    out = _embedding_sum(gidx, cidx, gtab, ctab)
    return out.reshape(b, s, emb)
